# baseline (speedup 1.0000x reference)
import jax
import jax.numpy as jnp
from jax import lax
from jax.experimental import pallas as pl
from jax.experimental.pallas import tpu as pltpu

_BM = 200


def _gcn_body(seq_ref, w_ref, adj0_ref, adj1_ref, b_ref, a_ref, out_ref, fts_ref):
    @pl.when(pl.program_id(0) == 0)
    def _():
        fts_ref[...] = lax.dot_general(
            seq_ref[...], w_ref[...], (((1,), (1,)), ((), ())),
            preferred_element_type=jnp.float32,
        )

    fts = fts_ref[...]
    bias = b_ref[...]
    a = a_ref[0]
    acc0 = jnp.dot(adj0_ref[0], fts, preferred_element_type=jnp.float32) + bias
    acc1 = jnp.dot(adj1_ref[0], fts, preferred_element_type=jnp.float32) + bias
    out_ref[0] = jnp.where(acc0 >= 0, acc0, a * acc0)
    out_ref[1] = jnp.where(acc1 >= 0, acc1, a * acc1)


def kernel(seq, adj, du, W, b, prelu_a):
    del du
    _, n, d_in = seq.shape
    d_out = W.shape[0]
    half = n // 2
    seq2 = seq.reshape(n, d_in)
    adj3 = adj.reshape(2, half, n)

    out = pl.pallas_call(
        _gcn_body,
        grid=(half // _BM,),
        in_specs=[
            pl.BlockSpec((n, d_in), lambda i: (0, 0)),
            pl.BlockSpec((d_out, d_in), lambda i: (0, 0)),
            pl.BlockSpec((1, _BM, n), lambda i: (0, i, 0)),
            pl.BlockSpec((1, _BM, n), lambda i: (1, i, 0)),
            pl.BlockSpec((d_out,), lambda i: (0,)),
            pl.BlockSpec((1,), lambda i: (0,)),
        ],
        out_specs=pl.BlockSpec((2, _BM, d_out), lambda i: (0, i, 0)),
        out_shape=jax.ShapeDtypeStruct((2, half, d_out), jnp.float32),
        scratch_shapes=[pltpu.VMEM((n, d_out), jnp.float32)],
    )(seq2, W, adj3, adj3, b, prelu_a)
    return out.reshape(1, n, d_out)


# speedup vs baseline: 1.0042x; 1.0042x over previous
import jax
import jax.numpy as jnp
from jax import lax
from jax.experimental import pallas as pl
from jax.experimental.pallas import tpu as pltpu

_BM = 400


def _gcn_body(seq_ref, w_ref, adj_ref, b_ref, a_ref, out_ref, fts_ref):
    @pl.when(pl.program_id(0) == 0)
    def _():
        fts_ref[...] = lax.dot_general(
            seq_ref[...], w_ref[...], (((1,), (1,)), ((), ())),
            preferred_element_type=jnp.float32,
        )

    acc = jnp.dot(adj_ref[...], fts_ref[...], preferred_element_type=jnp.float32)
    acc = acc + b_ref[...]
    out_ref[...] = jnp.where(acc >= 0, acc, a_ref[0] * acc)


def kernel(seq, adj, du, W, b, prelu_a):
    del du
    _, n, d_in = seq.shape
    d_out = W.shape[0]
    seq2 = seq.reshape(n, d_in)
    adj2 = adj.reshape(n, n)

    out = pl.pallas_call(
        _gcn_body,
        grid=(n // _BM,),
        in_specs=[
            pl.BlockSpec((n, d_in), lambda i: (0, 0)),
            pl.BlockSpec((d_out, d_in), lambda i: (0, 0)),
            pl.BlockSpec((_BM, n), lambda i: (i, 0)),
            pl.BlockSpec((d_out,), lambda i: (0,)),
            pl.BlockSpec((1,), lambda i: (0,)),
        ],
        out_specs=pl.BlockSpec((_BM, d_out), lambda i: (i, 0)),
        out_shape=jax.ShapeDtypeStruct((n, d_out), jnp.float32),
        scratch_shapes=[pltpu.VMEM((n, d_out), jnp.float32)],
    )(seq2, W, adj2, b, prelu_a)
    return out.reshape(1, n, d_out)


# speedup vs baseline: 1.0146x; 1.0104x over previous
import jax
import jax.numpy as jnp
from jax import lax
from jax.experimental import pallas as pl

_BM = 400


def _gcn_body(seq_ref, w_ref, adj_ref, b_ref, a_ref, out_ref):
    agg = jnp.dot(adj_ref[...], seq_ref[...], preferred_element_type=jnp.float32)
    acc = lax.dot_general(
        agg, w_ref[...], (((1,), (1,)), ((), ())),
        preferred_element_type=jnp.float32,
    )
    acc = acc + b_ref[...]
    out_ref[...] = jnp.where(acc >= 0, acc, a_ref[0] * acc)


def kernel(seq, adj, du, W, b, prelu_a):
    del du
    _, n, d_in = seq.shape
    d_out = W.shape[0]
    seq2 = seq.reshape(n, d_in)
    adj2 = adj.reshape(n, n)

    out = pl.pallas_call(
        _gcn_body,
        grid=(n // _BM,),
        in_specs=[
            pl.BlockSpec((n, d_in), lambda i: (0, 0)),
            pl.BlockSpec((d_out, d_in), lambda i: (0, 0)),
            pl.BlockSpec((_BM, n), lambda i: (i, 0)),
            pl.BlockSpec((d_out,), lambda i: (0,)),
            pl.BlockSpec((1,), lambda i: (0,)),
        ],
        out_specs=pl.BlockSpec((_BM, d_out), lambda i: (i, 0)),
        out_shape=jax.ShapeDtypeStruct((n, d_out), jnp.float32),
    )(seq2, W, adj2, b, prelu_a)
    return out.reshape(1, n, d_out)
